# baseline (speedup 1.0000x reference)
import functools

import numpy as np
import jax
import jax.numpy as jnp
from jax import lax
from jax.experimental import pallas as pl
from jax.experimental.pallas import tpu as pltpu
from jax.experimental.pallas import tpu_sc as plsc

_NC = 2
_NS = 16


def _posenc_np(seq_len, d_model):
    position = np.arange(seq_len, dtype=np.float32)[:, None]
    div_term = np.exp(
        np.arange(0, d_model, 2, dtype=np.float32) * (-np.log(10000.0) / d_model)
    )
    pe = np.zeros((seq_len, d_model), dtype=np.float32)
    pe[:, 0::2] = np.sin(position * div_term)
    pe[:, 1::2] = np.cos(position * div_term)
    return pe


@functools.lru_cache(maxsize=None)
def _make_sc_gather(B, N, M, D):
    NBG, NMG = 8, 4
    assert B % NBG == 0 and M % NMG == 0 and D % 16 == 0
    BPW = B // NBG
    MS = M // NMG
    CS = MS // 2
    CSP = 128
    NCH = BPW * 2
    NG = CSP // 16
    JD = D // 16

    mesh = plsc.VectorSubcoreMesh(core_axis_name="c", subcore_axis_name="s")

    @functools.partial(
        pl.kernel,
        out_type=jax.ShapeDtypeStruct((B * M, D), jnp.float32),
        mesh=mesh,
        compiler_params=pltpu.CompilerParams(use_tc_tiling_on_sc=False,
                                             needs_layout_passes=False),
        scratch_types=[
            pltpu.VMEM((CSP, 14), jnp.float32),
            pltpu.VMEM((CSP, 14), jnp.float32),
            pltpu.VMEM((2, 128), jnp.int32),
            pltpu.VMEM((2, 128), jnp.int32),
            pltpu.VMEM((CSP, D), jnp.float32),
            pltpu.VMEM((CSP, D), jnp.float32),
            pltpu.VMEM((CSP, D), jnp.float32),
            pltpu.VMEM((CSP, D), jnp.float32),
            pltpu.SemaphoreType.DMA,
            pltpu.SemaphoreType.DMA,
            pltpu.SemaphoreType.DMA,
            pltpu.SemaphoreType.DMA,
            pltpu.SemaphoreType.DMA,
            pltpu.SemaphoreType.DMA,
            pltpu.SemaphoreType.DMA,
            pltpu.SemaphoreType.DMA,
        ],
    )
    def kern(cities, state, out,
             state_v0, state_v1, idx_v0, idx_v1,
             rows0_v0, rows0_v1, rows1_v0, rows1_v1,
             st_s0, st_s1, g0_s0, g0_s1, g1_s0, g1_s1, o_s0, o_s1):
        state_v = [state_v0, state_v1]
        idx_v = [idx_v0, idx_v1]
        rows0_v = [rows0_v0, rows0_v1]
        rows1_v = [rows1_v0, rows1_v1]
        st_s = [st_s0, st_s1]
        g0_s = [g0_s0, g0_s1]
        g1_s = [g1_s0, g1_s1]
        o_s = [o_s0, o_s1]

        cid = lax.axis_index("c")
        sid = lax.axis_index("s")
        wid = sid * _NC + cid
        bg = wid // NMG
        mg = wid % NMG
        b_lo = bg * BPW
        m_lo = mg * MS

        lane = lax.iota(jnp.int32, 16)

        def chunk_base(c):
            b = b_lo + c // 2
            return b * M + m_lo + (c % 2) * CS, b

        def fire_state(c, k):
            abase, _ = chunk_base(c)
            pltpu.async_copy(state.at[pl.ds(abase, CS)],
                             state_v[k].at[pl.ds(0, CS)], st_s[k])

        def wait_state(k):
            pltpu.make_async_copy(state.at[pl.ds(0, CS)],
                                  state_v[k].at[pl.ds(0, CS)], st_s[k]).wait()

        def build_idx(c, k):
            _, b = chunk_base(c)
            uni = jnp.full((16,), True)
            first0 = first1 = None
            for g in range(NG):
                rowv = jnp.minimum(g * 16 + lane, CS - 1)
                f0 = plsc.load_gather(state_v[k],
                                      [rowv, jnp.full((16,), 0, jnp.int32)])
                f1 = plsc.load_gather(state_v[k],
                                      [rowv, jnp.full((16,), 1, jnp.int32)])
                i0 = jnp.clip(f0.astype(jnp.int32), 0, N - 1) + b * N
                i1 = jnp.clip(f1.astype(jnp.int32), 0, N - 1) + b * N
                if g == 0:
                    first0 = i0[0]
                    first1 = i1[0]
                uni = uni & (i0 == jnp.full((16,), first0)) \
                          & (i1 == jnp.full((16,), first1))
                idx_v[k][0, pl.ds(g * 16, 16)] = i0
                idx_v[k][1, pl.ds(g * 16, 16)] = i1
            return jnp.all(uni), first0, first1

        def fire_gathers(k, uni, row0, row1):
            @pl.when(uni)
            def _():
                pltpu.async_copy(cities.at[pl.ds(row0, 1)],
                                 rows0_v[k].at[pl.ds(0, 1)], g0_s[k])
                pltpu.async_copy(cities.at[pl.ds(row1, 1)],
                                 rows1_v[k].at[pl.ds(0, 1)], g1_s[k])

            @pl.when(jnp.logical_not(uni))
            def _():
                pltpu.async_copy(cities.at[idx_v[k].at[0]], rows0_v[k], g0_s[k])
                pltpu.async_copy(cities.at[idx_v[k].at[1]], rows1_v[k], g1_s[k])

        def wait_combine(k, uni):
            @pl.when(uni)
            def _():
                pltpu.make_async_copy(cities.at[pl.ds(0, 1)],
                                      rows0_v[k].at[pl.ds(0, 1)], g0_s[k]).wait()
                pltpu.make_async_copy(cities.at[pl.ds(0, 1)],
                                      rows1_v[k].at[pl.ds(0, 1)], g1_s[k]).wait()
                r0 = rows0_v[k]
                r1 = rows1_v[k]
                s = [r0[0, pl.ds(j * 16, 16)] + r1[0, pl.ds(j * 16, 16)]
                     for j in range(JD)]

                @plsc.parallel_loop(0, CS, unroll=4)
                def comb(a):
                    for j in range(JD):
                        r0[a, pl.ds(j * 16, 16)] = s[j]

            @pl.when(jnp.logical_not(uni))
            def _():
                pltpu.make_async_copy(cities.at[idx_v[k].at[0]], rows0_v[k],
                                      g0_s[k]).wait()
                pltpu.make_async_copy(cities.at[idx_v[k].at[1]], rows1_v[k],
                                      g1_s[k]).wait()
                r0 = rows0_v[k]
                r1 = rows1_v[k]

                @plsc.parallel_loop(0, CS, unroll=4)
                def comb(a):
                    for j in range(JD):
                        sl = pl.ds(j * 16, 16)
                        r0[a, sl] = r0[a, sl] + r1[a, sl]

        def fire_out(c, k):
            abase, _ = chunk_base(c)
            pltpu.async_copy(rows0_v[k].at[pl.ds(0, CS)],
                             out.at[pl.ds(abase, CS)], o_s[k])

        def wait_out(k):
            pltpu.make_async_copy(rows0_v[k].at[pl.ds(0, CS)],
                                  out.at[pl.ds(0, CS)], o_s[k]).wait()

        fire_state(0, 0)
        fire_state(1, 1)
        wait_state(0)
        u0, a0, b0 = build_idx(0, 0)
        fire_gathers(0, u0, a0, b0)

        def t_body(t, carry):
            fl = [carry[0], carry[1]]
            for k in (0, 1):
                c = 2 * t + k
                j = k ^ 1
                wait_state(j)
                uj, raj, rbj = build_idx(c + 1, j)
                fire_state(c + 2, k)

                @pl.when(c >= 1)
                def _():
                    wait_out(j)
                fire_gathers(j, uj, raj, rbj)

                wait_combine(k, fl[k])
                fire_out(c, k)
                fl[j] = uj
            return (fl[0], fl[1])

        carry = lax.fori_loop(0, NCH // 2 - 1, t_body,
                              (u0, jnp.full((), False)))

        wait_state(1)
        ut, rat, rbt = build_idx(NCH - 1, 1)
        wait_out(1)
        fire_gathers(1, ut, rat, rbt)
        wait_combine(0, carry[0])
        fire_out(NCH - 2, 0)
        wait_combine(1, ut)
        fire_out(NCH - 1, 1)
        wait_out(0)
        wait_out(1)

    return kern


@functools.lru_cache(maxsize=None)
def _make_tc_combine(B, M, D):
    grid = (B,)

    def body(g_ref, s_ref, gr_ref, bps_ref, w_ref, pe_ref, o_ref):
        lin = lax.dot_general(s_ref[0], w_ref[...],
                              (((1,), (0,)), ((), ())),
                              preferred_element_type=jnp.float32)
        o_ref[0] = g_ref[0] + lin + pe_ref[...] + gr_ref[0] + bps_ref[0]

    return pl.pallas_call(
        body,
        grid=grid,
        in_specs=[
            pl.BlockSpec((1, M, D), lambda b: (b, 0, 0)),
            pl.BlockSpec((1, M, 8), lambda b: (b, 0, 0)),
            pl.BlockSpec((1, 1, D), lambda b: (b, 0, 0)),
            pl.BlockSpec((1, D), lambda b: (0, 0)),
            pl.BlockSpec((8, D), lambda b: (0, 0)),
            pl.BlockSpec((M, D), lambda b: (0, 0)),
        ],
        out_specs=pl.BlockSpec((1, M, D), lambda b: (b, 0, 0)),
        out_shape=jax.ShapeDtypeStruct((B, M, D), jnp.float32),
    )


def kernel(cities_embed, graph_embed, agent_state, W_dc, W_nc, W_ps, b_ps):
    B, N, D = cities_embed.shape
    M = agent_state.shape[1]
    cities = cities_embed.reshape(B * N, D)
    state = agent_state.reshape(B * M, 14)

    gsum = _make_sc_gather(B, N, M, D)(cities, state)

    w8 = jnp.concatenate(
        [W_dc, W_nc, W_ps, jnp.zeros((D, 2), jnp.float32)], axis=1).T
    pe = jnp.asarray(_posenc_np(M, D))
    out = _make_tc_combine(B, M, D)(
        gsum.reshape(B, M, D), agent_state[:, :, 2:10], graph_embed,
        b_ps.reshape(1, D), w8, pe)
    return out


# speedup vs baseline: 1.6977x; 1.6977x over previous
import functools

import numpy as np
import jax
import jax.numpy as jnp
from jax import lax
from jax.experimental import pallas as pl
from jax.experimental.pallas import tpu as pltpu
from jax.experimental.pallas import tpu_sc as plsc

_NC = 2
_NS = 16


def _posenc_np(seq_len, d_model):
    position = np.arange(seq_len, dtype=np.float32)[:, None]
    div_term = np.exp(
        np.arange(0, d_model, 2, dtype=np.float32) * (-np.log(10000.0) / d_model)
    )
    pe = np.zeros((seq_len, d_model), dtype=np.float32)
    pe[:, 0::2] = np.sin(position * div_term)
    pe[:, 1::2] = np.cos(position * div_term)
    return pe


@functools.lru_cache(maxsize=None)
def _make_sc_gather(B, N, M, D):
    NBG, NMG = 8, 4
    assert B % NBG == 0 and M % NMG == 0 and D % 16 == 0
    BPW = B // NBG
    MS = M // NMG
    CS = MS // 2
    CSP = 128
    NCH = BPW * 2
    NG = CSP // 16
    JD = D // 16

    mesh = plsc.VectorSubcoreMesh(core_axis_name="c", subcore_axis_name="s")

    @functools.partial(
        pl.kernel,
        out_type=jax.ShapeDtypeStruct((B * M, D), jnp.float32),
        mesh=mesh,
        compiler_params=pltpu.CompilerParams(use_tc_tiling_on_sc=False,
                                             needs_layout_passes=False),
        scratch_types=[
            pltpu.VMEM((2, CSP + 8), jnp.float32),
            pltpu.VMEM((2, CSP + 8), jnp.float32),
            pltpu.VMEM((2, 128), jnp.int32),
            pltpu.VMEM((2, 128), jnp.int32),
            pltpu.VMEM((CSP, D), jnp.float32),
            pltpu.VMEM((CSP, D), jnp.float32),
            pltpu.VMEM((CSP, D), jnp.float32),
            pltpu.VMEM((CSP, D), jnp.float32),
            pltpu.SemaphoreType.DMA,
            pltpu.SemaphoreType.DMA,
            pltpu.SemaphoreType.DMA,
            pltpu.SemaphoreType.DMA,
            pltpu.SemaphoreType.DMA,
            pltpu.SemaphoreType.DMA,
            pltpu.SemaphoreType.DMA,
            pltpu.SemaphoreType.DMA,
        ],
    )
    def kern(cities, state, out,
             state_v0, state_v1, idx_v0, idx_v1,
             rows0_v0, rows0_v1, rows1_v0, rows1_v1,
             st_s0, st_s1, g0_s0, g0_s1, g1_s0, g1_s1, o_s0, o_s1):
        state_v = [state_v0, state_v1]
        idx_v = [idx_v0, idx_v1]
        rows0_v = [rows0_v0, rows0_v1]
        rows1_v = [rows1_v0, rows1_v1]
        st_s = [st_s0, st_s1]
        g0_s = [g0_s0, g0_s1]
        g1_s = [g1_s0, g1_s1]
        o_s = [o_s0, o_s1]

        cid = lax.axis_index("c")
        sid = lax.axis_index("s")
        wid = sid * _NC + cid
        bg = wid // NMG
        mg = wid % NMG
        b_lo = bg * BPW
        m_lo = mg * MS

        lane = lax.iota(jnp.int32, 16)

        def chunk_base(c):
            b = b_lo + c // 2
            return b * M + m_lo + (c % 2) * CS, b

        def fire_state(c, k):
            abase, _ = chunk_base(c)
            a0 = (abase // 8) * 8
            pltpu.async_copy(state.at[:, pl.ds(a0, CSP + 8)],
                             state_v[k], st_s[k])

        def wait_state(k):
            pltpu.make_async_copy(state.at[:, pl.ds(0, CSP + 8)],
                                  state_v[k], st_s[k]).wait()

        def build_idx(c, k):
            abase, b = chunk_base(c)
            shift = abase % 8
            uni = jnp.full((16,), True)
            first0 = first1 = None
            for g in range(NG):
                sl = pl.ds(shift + g * 16, 16)
                f0 = state_v[k][0, sl]
                f1 = state_v[k][1, sl]
                i0 = jnp.clip(f0.astype(jnp.int32), 0, N - 1) + b * N
                i1 = jnp.clip(f1.astype(jnp.int32), 0, N - 1) + b * N
                if g == 0:
                    first0 = i0[0]
                    first1 = i1[0]
                tail = (g * 16 + lane) >= CS
                uni = uni & (((i0 == jnp.full((16,), first0))
                              & (i1 == jnp.full((16,), first1))) | tail)
                slo = pl.ds(g * 16, 16)
                idx_v[k][0, slo] = i0
                idx_v[k][1, slo] = i1
            return jnp.all(uni), first0, first1

        def fire_gathers(k, uni, row0, row1):
            @pl.when(uni)
            def _():
                pltpu.async_copy(cities.at[pl.ds(row0, 1)],
                                 rows0_v[k].at[pl.ds(0, 1)], g0_s[k])
                pltpu.async_copy(cities.at[pl.ds(row1, 1)],
                                 rows1_v[k].at[pl.ds(0, 1)], g1_s[k])

            @pl.when(jnp.logical_not(uni))
            def _():
                pltpu.async_copy(cities.at[idx_v[k].at[0]], rows0_v[k], g0_s[k])
                pltpu.async_copy(cities.at[idx_v[k].at[1]], rows1_v[k], g1_s[k])

        def wait_combine(k, uni):
            @pl.when(uni)
            def _():
                pltpu.make_async_copy(cities.at[pl.ds(0, 1)],
                                      rows0_v[k].at[pl.ds(0, 1)], g0_s[k]).wait()
                pltpu.make_async_copy(cities.at[pl.ds(0, 1)],
                                      rows1_v[k].at[pl.ds(0, 1)], g1_s[k]).wait()
                r0 = rows0_v[k]
                r1 = rows1_v[k]
                s = [r0[0, pl.ds(j * 16, 16)] + r1[0, pl.ds(j * 16, 16)]
                     for j in range(JD)]

                @plsc.parallel_loop(0, CS, unroll=4)
                def comb(a):
                    for j in range(JD):
                        r0[a, pl.ds(j * 16, 16)] = s[j]

            @pl.when(jnp.logical_not(uni))
            def _():
                pltpu.make_async_copy(cities.at[idx_v[k].at[0]], rows0_v[k],
                                      g0_s[k]).wait()
                pltpu.make_async_copy(cities.at[idx_v[k].at[1]], rows1_v[k],
                                      g1_s[k]).wait()
                r0 = rows0_v[k]
                r1 = rows1_v[k]

                @plsc.parallel_loop(0, CS, unroll=4)
                def comb(a):
                    for j in range(JD):
                        sl = pl.ds(j * 16, 16)
                        r0[a, sl] = r0[a, sl] + r1[a, sl]

        def fire_out(c, k):
            abase, _ = chunk_base(c)
            pltpu.async_copy(rows0_v[k].at[pl.ds(0, CS)],
                             out.at[pl.ds(abase, CS)], o_s[k])

        def wait_out(k):
            pltpu.make_async_copy(rows0_v[k].at[pl.ds(0, CS)],
                                  out.at[pl.ds(0, CS)], o_s[k]).wait()

        fire_state(0, 0)
        fire_state(1, 1)
        wait_state(0)
        u0, a0, b0 = build_idx(0, 0)
        fire_gathers(0, u0, a0, b0)

        def t_body(t, carry):
            fl = [carry[0], carry[1]]
            for k in (0, 1):
                c = 2 * t + k
                j = k ^ 1
                wait_state(j)
                uj, raj, rbj = build_idx(c + 1, j)
                fire_state(c + 2, k)

                @pl.when(c >= 1)
                def _():
                    wait_out(j)
                fire_gathers(j, uj, raj, rbj)

                wait_combine(k, fl[k])
                fire_out(c, k)
                fl[j] = uj
            return (fl[0], fl[1])

        carry = lax.fori_loop(0, NCH // 2 - 1, t_body,
                              (u0, jnp.full((), False)))

        wait_state(1)
        ut, rat, rbt = build_idx(NCH - 1, 1)
        wait_out(1)
        fire_gathers(1, ut, rat, rbt)
        wait_combine(0, carry[0])
        fire_out(NCH - 2, 0)
        wait_combine(1, ut)
        fire_out(NCH - 1, 1)
        wait_out(0)
        wait_out(1)

    return kern


@functools.lru_cache(maxsize=None)
def _make_tc_combine(B, M, D):
    grid = (B,)

    def body(g_ref, s_ref, gr_ref, bps_ref, w_ref, pe_ref, o_ref):
        lin = lax.dot_general(s_ref[0], w_ref[...],
                              (((0,), (0,)), ((), ())),
                              preferred_element_type=jnp.float32)
        o_ref[0] = g_ref[0] + lin + pe_ref[...] + gr_ref[0] + bps_ref[0]

    return pl.pallas_call(
        body,
        grid=grid,
        in_specs=[
            pl.BlockSpec((1, M, D), lambda b: (b, 0, 0)),
            pl.BlockSpec((1, 8, M), lambda b: (b, 0, 0)),
            pl.BlockSpec((1, 1, D), lambda b: (b, 0, 0)),
            pl.BlockSpec((1, D), lambda b: (0, 0)),
            pl.BlockSpec((8, D), lambda b: (0, 0)),
            pl.BlockSpec((M, D), lambda b: (0, 0)),
        ],
        out_specs=pl.BlockSpec((1, M, D), lambda b: (b, 0, 0)),
        out_shape=jax.ShapeDtypeStruct((B, M, D), jnp.float32),
    )


def kernel(cities_embed, graph_embed, agent_state, W_dc, W_nc, W_ps, b_ps):
    B, N, D = cities_embed.shape
    M = agent_state.shape[1]
    cities = cities_embed.reshape(B * N, D)
    state01 = jnp.pad(agent_state[:, :, :2].reshape(B * M, 2).T,
                      ((0, 0), (0, 8)))

    gsum = _make_sc_gather(B, N, M, D)(cities, state01)

    w8 = jnp.concatenate(
        [W_dc, W_nc, W_ps, jnp.zeros((D, 2), jnp.float32)], axis=1).T
    pe = jnp.asarray(_posenc_np(M, D))
    s8t = agent_state[:, :, 2:10].transpose(0, 2, 1)
    out = _make_tc_combine(B, M, D)(
        gsum.reshape(B, M, D), s8t, graph_embed,
        b_ps.reshape(1, D), w8, pe)
    return out


# speedup vs baseline: 2.7610x; 1.6264x over previous
import functools

import numpy as np
import jax
import jax.numpy as jnp
from jax import lax
from jax.experimental import pallas as pl
from jax.experimental.pallas import tpu as pltpu
from jax.experimental.pallas import tpu_sc as plsc

_NC = 2
_NS = 16


def _posenc_np(seq_len, d_model):
    position = np.arange(seq_len, dtype=np.float32)[:, None]
    div_term = np.exp(
        np.arange(0, d_model, 2, dtype=np.float32) * (-np.log(10000.0) / d_model)
    )
    pe = np.zeros((seq_len, d_model), dtype=np.float32)
    pe[:, 0::2] = np.sin(position * div_term)
    pe[:, 1::2] = np.cos(position * div_term)
    return pe


@functools.lru_cache(maxsize=None)
def _make_sc_gather(B, N, M, D):
    NBG, NMG = 8, 4
    assert B % NBG == 0 and M % NMG == 0 and D % 16 == 0
    BPW = B // NBG
    MS = M // NMG
    CS = MS // 2
    CSP = 128
    NCH = BPW * 2
    NG = CSP // 16
    JD = D // 16

    mesh = plsc.VectorSubcoreMesh(core_axis_name="c", subcore_axis_name="s")

    @functools.partial(
        pl.kernel,
        out_type=jax.ShapeDtypeStruct((B * M, D), jnp.float32),
        mesh=mesh,
        compiler_params=pltpu.CompilerParams(use_tc_tiling_on_sc=False,
                                             needs_layout_passes=False),
        scratch_types=[
            pltpu.VMEM((2, CSP + 8), jnp.float32),
            pltpu.VMEM((2, CSP + 8), jnp.float32),
            pltpu.VMEM((2, 128), jnp.int32),
            pltpu.VMEM((2, 128), jnp.int32),
            pltpu.VMEM((CSP, D), jnp.float32),
            pltpu.VMEM((CSP, D), jnp.float32),
            pltpu.VMEM((CSP, D), jnp.float32),
            pltpu.VMEM((CSP, D), jnp.float32),
            pltpu.SemaphoreType.DMA,
            pltpu.SemaphoreType.DMA,
            pltpu.SemaphoreType.DMA,
            pltpu.SemaphoreType.DMA,
            pltpu.SemaphoreType.DMA,
            pltpu.SemaphoreType.DMA,
            pltpu.SemaphoreType.DMA,
            pltpu.SemaphoreType.DMA,
        ],
    )
    def kern(cities, state, out,
             state_v0, state_v1, idx_v0, idx_v1,
             rows0_v0, rows0_v1, rows1_v0, rows1_v1,
             st_s0, st_s1, g0_s0, g0_s1, g1_s0, g1_s1, o_s0, o_s1):
        state_v = [state_v0, state_v1]
        idx_v = [idx_v0, idx_v1]
        rows0_v = [rows0_v0, rows0_v1]
        rows1_v = [rows1_v0, rows1_v1]
        st_s = [st_s0, st_s1]
        g0_s = [g0_s0, g0_s1]
        g1_s = [g1_s0, g1_s1]
        o_s = [o_s0, o_s1]

        cid = lax.axis_index("c")
        sid = lax.axis_index("s")
        wid = sid * _NC + cid
        bg = wid // NMG
        mg = wid % NMG
        b_lo = bg * BPW
        m_lo = mg * MS

        lane = lax.iota(jnp.int32, 16)

        def chunk_base(c):
            b = b_lo + c // 2
            return b * M + m_lo + (c % 2) * CS, b

        def fire_state(c, k):
            abase, _ = chunk_base(c)
            a0 = (abase // 8) * 8
            pltpu.async_copy(state.at[:, pl.ds(a0, CSP + 8)],
                             state_v[k], st_s[k])

        def wait_state(k):
            pltpu.make_async_copy(state.at[:, pl.ds(0, CSP + 8)],
                                  state_v[k], st_s[k]).wait()

        def build_idx(c, k):
            abase, b = chunk_base(c)
            shift = abase % 8
            uni = jnp.full((16,), True)
            first0 = first1 = None
            for g in range(NG):
                sl = pl.ds(shift + g * 16, 16)
                f0 = state_v[k][0, sl]
                f1 = state_v[k][1, sl]
                i0 = jnp.clip(f0.astype(jnp.int32), 0, N - 1) + b * N
                i1 = jnp.clip(f1.astype(jnp.int32), 0, N - 1) + b * N
                if g == 0:
                    first0 = i0[0]
                    first1 = i1[0]
                tail = (g * 16 + lane) >= CS
                uni = uni & (((i0 == jnp.full((16,), first0))
                              & (i1 == jnp.full((16,), first1))) | tail)
                slo = pl.ds(g * 16, 16)
                idx_v[k][0, slo] = i0
                idx_v[k][1, slo] = i1
            return jnp.all(uni), first0, first1

        def fire_gathers(k, uni, row0, row1):
            @pl.when(uni)
            def _():
                pltpu.async_copy(cities.at[pl.ds(row0, 1)],
                                 rows0_v[k].at[pl.ds(0, 1)], g0_s[k])
                pltpu.async_copy(cities.at[pl.ds(row1, 1)],
                                 rows1_v[k].at[pl.ds(0, 1)], g1_s[k])

            @pl.when(jnp.logical_not(uni))
            def _():
                pltpu.async_copy(cities.at[idx_v[k].at[0]], rows0_v[k], g0_s[k])
                pltpu.async_copy(cities.at[idx_v[k].at[1]], rows1_v[k], g1_s[k])

        def wait_combine(k, uni):
            @pl.when(uni)
            def _():
                pltpu.make_async_copy(cities.at[pl.ds(0, 1)],
                                      rows0_v[k].at[pl.ds(0, 1)], g0_s[k]).wait()
                pltpu.make_async_copy(cities.at[pl.ds(0, 1)],
                                      rows1_v[k].at[pl.ds(0, 1)], g1_s[k]).wait()
                r0 = rows0_v[k]
                r1 = rows1_v[k]
                s = [r0[0, pl.ds(j * 16, 16)] + r1[0, pl.ds(j * 16, 16)]
                     for j in range(JD)]

                @plsc.parallel_loop(0, CS, unroll=4)
                def comb(a):
                    for j in range(JD):
                        r0[a, pl.ds(j * 16, 16)] = s[j]

            @pl.when(jnp.logical_not(uni))
            def _():
                pltpu.make_async_copy(cities.at[idx_v[k].at[0]], rows0_v[k],
                                      g0_s[k]).wait()
                pltpu.make_async_copy(cities.at[idx_v[k].at[1]], rows1_v[k],
                                      g1_s[k]).wait()
                r0 = rows0_v[k]
                r1 = rows1_v[k]

                @plsc.parallel_loop(0, CS, unroll=4)
                def comb(a):
                    for j in range(JD):
                        sl = pl.ds(j * 16, 16)
                        r0[a, sl] = r0[a, sl] + r1[a, sl]

        def fire_out(c, k):
            abase, _ = chunk_base(c)
            pltpu.async_copy(rows0_v[k].at[pl.ds(0, CS)],
                             out.at[pl.ds(abase, CS)], o_s[k])

        def wait_out(k):
            pltpu.make_async_copy(rows0_v[k].at[pl.ds(0, CS)],
                                  out.at[pl.ds(0, CS)], o_s[k]).wait()

        fire_state(0, 0)
        fire_state(1, 1)
        wait_state(0)
        u0, a0, b0 = build_idx(0, 0)
        fire_gathers(0, u0, a0, b0)

        def t_body(t, carry):
            fl = [carry[0], carry[1]]
            for k in (0, 1):
                c = 2 * t + k
                j = k ^ 1
                wait_state(j)
                uj, raj, rbj = build_idx(c + 1, j)
                fire_state(c + 2, k)

                @pl.when(c >= 1)
                def _():
                    wait_out(j)
                fire_gathers(j, uj, raj, rbj)

                wait_combine(k, fl[k])
                fire_out(c, k)
                fl[j] = uj
            return (fl[0], fl[1])

        carry = lax.fori_loop(0, NCH // 2 - 1, t_body,
                              (u0, jnp.full((), False)))

        wait_state(1)
        ut, rat, rbt = build_idx(NCH - 1, 1)
        wait_out(1)
        fire_gathers(1, ut, rat, rbt)
        wait_combine(0, carry[0])
        fire_out(NCH - 2, 0)
        wait_combine(1, ut)
        fire_out(NCH - 1, 1)
        wait_out(0)
        wait_out(1)

    return kern


@functools.lru_cache(maxsize=None)
def _make_tc_combine(B, M, D):
    BB = 8
    grid = (B // BB,)

    def body(g_ref, s_ref, gr_ref, bps_ref, w_ref, pe_ref, o_ref):
        lin = lax.dot_general(s_ref[...], w_ref[...],
                              (((1,), (0,)), ((), ())),
                              preferred_element_type=jnp.float32)
        o_ref[...] = (g_ref[...] + lin + pe_ref[...][None]
                      + gr_ref[...] + bps_ref[...][None])

    return pl.pallas_call(
        body,
        grid=grid,
        in_specs=[
            pl.BlockSpec((BB, M, D), lambda b: (b, 0, 0)),
            pl.BlockSpec((BB, 8, M), lambda b: (b, 0, 0)),
            pl.BlockSpec((BB, 1, D), lambda b: (b, 0, 0)),
            pl.BlockSpec((1, D), lambda b: (0, 0)),
            pl.BlockSpec((8, D), lambda b: (0, 0)),
            pl.BlockSpec((M, D), lambda b: (0, 0)),
        ],
        out_specs=pl.BlockSpec((BB, M, D), lambda b: (b, 0, 0)),
        out_shape=jax.ShapeDtypeStruct((B, M, D), jnp.float32),
    )


def kernel(cities_embed, graph_embed, agent_state, W_dc, W_nc, W_ps, b_ps):
    B, N, D = cities_embed.shape
    M = agent_state.shape[1]
    cities = cities_embed.reshape(B * N, D)
    state01 = jnp.pad(agent_state[:, :, :2].reshape(B * M, 2).T,
                      ((0, 0), (0, 8)))

    gsum = _make_sc_gather(B, N, M, D)(cities, state01)

    w8 = jnp.concatenate(
        [W_dc, W_nc, W_ps, jnp.zeros((D, 2), jnp.float32)], axis=1).T
    pe = jnp.asarray(_posenc_np(M, D))
    s8t = agent_state[:, :, 2:10].transpose(0, 2, 1)
    out = _make_tc_combine(B, M, D)(
        gsum.reshape(B, M, D), s8t, graph_embed,
        b_ps.reshape(1, D), w8, pe)
    return out


# speedup vs baseline: 2.8006x; 1.0143x over previous
import functools

import numpy as np
import jax
import jax.numpy as jnp
from jax import lax
from jax.experimental import pallas as pl
from jax.experimental.pallas import tpu as pltpu
from jax.experimental.pallas import tpu_sc as plsc

_NC = 2
_NS = 16


def _posenc_np(seq_len, d_model):
    position = np.arange(seq_len, dtype=np.float32)[:, None]
    div_term = np.exp(
        np.arange(0, d_model, 2, dtype=np.float32) * (-np.log(10000.0) / d_model)
    )
    pe = np.zeros((seq_len, d_model), dtype=np.float32)
    pe[:, 0::2] = np.sin(position * div_term)
    pe[:, 1::2] = np.cos(position * div_term)
    return pe


@functools.lru_cache(maxsize=None)
def _make_sc_gather(B, N, M, D):
    NBG, NMG = 8, 4
    assert B % NBG == 0 and M % NMG == 0 and D % 16 == 0
    BPW = B // NBG
    MS = M // NMG
    CS = MS // 2
    CSP = 128
    NCH = BPW * 2
    NG = CSP // 16
    JD = D // 16

    mesh = plsc.VectorSubcoreMesh(core_axis_name="c", subcore_axis_name="s")

    @functools.partial(
        pl.kernel,
        out_type=jax.ShapeDtypeStruct((B * M, D), jnp.float32),
        mesh=mesh,
        compiler_params=pltpu.CompilerParams(use_tc_tiling_on_sc=False,
                                             needs_layout_passes=False),
        scratch_types=[
            pltpu.VMEM((2, CSP + 8), jnp.float32),
            pltpu.VMEM((2, CSP + 8), jnp.float32),
            pltpu.VMEM((2, 128), jnp.int32),
            pltpu.VMEM((2, 128), jnp.int32),
            pltpu.VMEM((CSP, D), jnp.float32),
            pltpu.VMEM((CSP, D), jnp.float32),
            pltpu.VMEM((CSP, D), jnp.float32),
            pltpu.VMEM((CSP, D), jnp.float32),
            pltpu.SemaphoreType.DMA,
            pltpu.SemaphoreType.DMA,
            pltpu.SemaphoreType.DMA,
            pltpu.SemaphoreType.DMA,
            pltpu.SemaphoreType.DMA,
            pltpu.SemaphoreType.DMA,
            pltpu.SemaphoreType.DMA,
            pltpu.SemaphoreType.DMA,
        ],
    )
    def kern(cities, state, out,
             state_v0, state_v1, idx_v0, idx_v1,
             rows0_v0, rows0_v1, rows1_v0, rows1_v1,
             st_s0, st_s1, g0_s0, g0_s1, g1_s0, g1_s1, o_s0, o_s1):
        state_v = [state_v0, state_v1]
        idx_v = [idx_v0, idx_v1]
        rows0_v = [rows0_v0, rows0_v1]
        rows1_v = [rows1_v0, rows1_v1]
        st_s = [st_s0, st_s1]
        g0_s = [g0_s0, g0_s1]
        g1_s = [g1_s0, g1_s1]
        o_s = [o_s0, o_s1]

        cid = lax.axis_index("c")
        sid = lax.axis_index("s")
        wid = sid * _NC + cid
        bg = wid // NMG
        mg = wid % NMG
        b_lo = bg * BPW
        m_lo = mg * MS

        lane = lax.iota(jnp.int32, 16)

        def chunk_base(c):
            b = b_lo + c // 2
            return b * M + m_lo + (c % 2) * CS, b

        def fire_state(c, k):
            abase, _ = chunk_base(c)
            a0 = (abase // 8) * 8
            pltpu.async_copy(state.at[:, pl.ds(a0, CSP + 8)],
                             state_v[k], st_s[k])

        def wait_state(k):
            pltpu.make_async_copy(state.at[:, pl.ds(0, CSP + 8)],
                                  state_v[k], st_s[k]).wait()

        def build_idx(c, k):
            abase, b = chunk_base(c)
            shift = abase % 8
            uni = jnp.full((16,), True)
            first0 = first1 = None
            for g in range(NG):
                sl = pl.ds(shift + g * 16, 16)
                f0 = state_v[k][0, sl]
                f1 = state_v[k][1, sl]
                i0 = jnp.clip(f0.astype(jnp.int32), 0, N - 1) + b * N
                i1 = jnp.clip(f1.astype(jnp.int32), 0, N - 1) + b * N
                if g == 0:
                    first0 = i0[0]
                    first1 = i1[0]
                tail = (g * 16 + lane) >= CS
                uni = uni & (((i0 == jnp.full((16,), first0))
                              & (i1 == jnp.full((16,), first1))) | tail)
                slo = pl.ds(g * 16, 16)
                idx_v[k][0, slo] = i0
                idx_v[k][1, slo] = i1
            return jnp.all(uni), first0, first1

        def fire_gathers(k, uni, row0, row1):
            @pl.when(uni)
            def _():
                pltpu.async_copy(cities.at[pl.ds(row0, 1)],
                                 rows0_v[k].at[pl.ds(0, 1)], g0_s[k])
                pltpu.async_copy(cities.at[pl.ds(row1, 1)],
                                 rows1_v[k].at[pl.ds(0, 1)], g1_s[k])

            @pl.when(jnp.logical_not(uni))
            def _():
                pltpu.async_copy(cities.at[idx_v[k].at[0]], rows0_v[k], g0_s[k])
                pltpu.async_copy(cities.at[idx_v[k].at[1]], rows1_v[k], g1_s[k])

        def wait_combine(k, uni):
            @pl.when(uni)
            def _():
                pltpu.make_async_copy(cities.at[pl.ds(0, 1)],
                                      rows0_v[k].at[pl.ds(0, 1)], g0_s[k]).wait()
                pltpu.make_async_copy(cities.at[pl.ds(0, 1)],
                                      rows1_v[k].at[pl.ds(0, 1)], g1_s[k]).wait()
                r0 = rows0_v[k]
                r1 = rows1_v[k]
                s = [r0[0, pl.ds(j * 16, 16)] + r1[0, pl.ds(j * 16, 16)]
                     for j in range(JD)]

                @plsc.parallel_loop(0, CS, unroll=4)
                def comb(a):
                    for j in range(JD):
                        r0[a, pl.ds(j * 16, 16)] = s[j]

            @pl.when(jnp.logical_not(uni))
            def _():
                pltpu.make_async_copy(cities.at[idx_v[k].at[0]], rows0_v[k],
                                      g0_s[k]).wait()
                pltpu.make_async_copy(cities.at[idx_v[k].at[1]], rows1_v[k],
                                      g1_s[k]).wait()
                r0 = rows0_v[k]
                r1 = rows1_v[k]

                @plsc.parallel_loop(0, CS, unroll=4)
                def comb(a):
                    for j in range(JD):
                        sl = pl.ds(j * 16, 16)
                        r0[a, sl] = r0[a, sl] + r1[a, sl]

        def fire_out(c, k):
            abase, _ = chunk_base(c)
            pltpu.async_copy(rows0_v[k].at[pl.ds(0, CS)],
                             out.at[pl.ds(abase, CS)], o_s[k])

        def wait_out(k):
            pltpu.make_async_copy(rows0_v[k].at[pl.ds(0, CS)],
                                  out.at[pl.ds(0, CS)], o_s[k]).wait()

        fire_state(0, 0)
        fire_state(1, 1)
        wait_state(0)
        u0, a0, b0 = build_idx(0, 0)
        fire_gathers(0, u0, a0, b0)

        def t_body(t, carry):
            fl = [carry[0], carry[1]]
            for k in (0, 1):
                c = 2 * t + k
                j = k ^ 1
                wait_state(j)
                uj, raj, rbj = build_idx(c + 1, j)
                fire_state(c + 2, k)

                @pl.when(c >= 1)
                def _():
                    wait_out(j)
                fire_gathers(j, uj, raj, rbj)

                wait_combine(k, fl[k])
                fire_out(c, k)
                fl[j] = uj
            return (fl[0], fl[1])

        carry = lax.fori_loop(0, NCH // 2 - 1, t_body,
                              (u0, jnp.full((), False)))

        wait_state(1)
        ut, rat, rbt = build_idx(NCH - 1, 1)
        wait_out(1)
        fire_gathers(1, ut, rat, rbt)
        wait_combine(0, carry[0])
        fire_out(NCH - 2, 0)
        wait_combine(1, ut)
        fire_out(NCH - 1, 1)
        wait_out(0)
        wait_out(1)

    return kern


@functools.lru_cache(maxsize=None)
def _make_tc_combine(B, M, D):
    BB = 16
    grid = (B // BB,)

    def body(g_ref, s_ref, gr_ref, bps_ref, w_ref, pe_ref, o_ref):
        lin = lax.dot_general(s_ref[...], w_ref[...],
                              (((1,), (0,)), ((), ())),
                              preferred_element_type=jnp.float32)
        o_ref[...] = (g_ref[...] + lin + pe_ref[...][None]
                      + gr_ref[...] + bps_ref[...][None])

    return pl.pallas_call(
        body,
        grid=grid,
        in_specs=[
            pl.BlockSpec((BB, M, D), lambda b: (b, 0, 0)),
            pl.BlockSpec((BB, 8, M), lambda b: (b, 0, 0)),
            pl.BlockSpec((BB, 1, D), lambda b: (b, 0, 0)),
            pl.BlockSpec((1, D), lambda b: (0, 0)),
            pl.BlockSpec((8, D), lambda b: (0, 0)),
            pl.BlockSpec((M, D), lambda b: (0, 0)),
        ],
        out_specs=pl.BlockSpec((BB, M, D), lambda b: (b, 0, 0)),
        out_shape=jax.ShapeDtypeStruct((B, M, D), jnp.float32),
    )


def kernel(cities_embed, graph_embed, agent_state, W_dc, W_nc, W_ps, b_ps):
    B, N, D = cities_embed.shape
    M = agent_state.shape[1]
    cities = cities_embed.reshape(B * N, D)
    state01 = jnp.pad(agent_state[:, :, :2].reshape(B * M, 2).T,
                      ((0, 0), (0, 8)))

    gsum = _make_sc_gather(B, N, M, D)(cities, state01)

    w8 = jnp.concatenate(
        [W_dc, W_nc, W_ps, jnp.zeros((D, 2), jnp.float32)], axis=1).T
    pe = jnp.asarray(_posenc_np(M, D))
    s8t = agent_state[:, :, 2:10].transpose(0, 2, 1)
    out = _make_tc_combine(B, M, D)(
        gsum.reshape(B, M, D), s8t, graph_embed,
        b_ps.reshape(1, D), w8, pe)
    return out
